# baseline (speedup 1.0000x reference)
import functools

import jax
import jax.numpy as jnp
from jax import lax
from jax.experimental import pallas as pl
from jax.experimental.pallas import tpu as pltpu
from jax.experimental.pallas import tpu_sc as plsc

_B, _T, _D = 4096, 200, 64
_LANES = 16
_C0, _C1 = 128, _T - 128


@functools.cache
def _build_kernel():
    info = plsc.get_sparse_core_info()
    nc, ns = info.num_cores, info.num_subcores
    nw = nc * ns
    rows_per_w = _B // nw
    mesh = plsc.VectorSubcoreMesh(core_axis_name="c", subcore_axis_name="s")

    @functools.partial(
        pl.kernel,
        mesh=mesh,
        compiler_params=pltpu.CompilerParams(use_tc_tiling_on_sc=False),
        out_type=jax.ShapeDtypeStruct((_B * _T, _D), jnp.float32),
        scratch_types=[
            pltpu.VMEM((_T,), jnp.int32),
            pltpu.VMEM((_T, _D), jnp.float32),
            pltpu.VMEM((_T, _D), jnp.float32),
            pltpu.SemaphoreType.DMA,
        ],
    )
    def k(codes_hbm, emb_hbm, pos_hbm, out_hbm, idx_v, rows_v, pos_v, sem):
        wid = lax.axis_index("s") * nc + lax.axis_index("c")
        pltpu.sync_copy(pos_hbm, pos_v)

        def row_body(r, carry):
            base = pl.multiple_of((wid * rows_per_w + r) * _T, 8)
            pltpu.sync_copy(codes_hbm.at[pl.ds(base, _T)], idx_v)
            g0 = pltpu.async_copy(
                emb_hbm.at[idx_v.at[pl.ds(0, _C0)]], rows_v.at[pl.ds(0, _C0)], sem)
            g1 = pltpu.async_copy(
                emb_hbm.at[idx_v.at[pl.ds(_C0, _C1)]], rows_v.at[pl.ds(_C0, _C1)], sem)
            g0.wait()
            g1.wait()

            def add_body(t, c):
                for j in range(_D // _LANES):
                    sl = pl.ds(j * _LANES, _LANES)
                    rows_v[t, sl] = rows_v[t, sl] + pos_v[t, sl]
                return c

            lax.fori_loop(0, _T, add_body, 0)
            pltpu.sync_copy(rows_v, out_hbm.at[pl.ds(base, _T)])
            return carry

        lax.fori_loop(0, rows_per_w, row_body, 0)

    return k


def kernel(motion_codes, motion_pad_mask, emb_weight, pos_weight):
    codes = motion_codes.reshape(-1).astype(jnp.int32)
    x = _build_kernel()(codes, emb_weight, pos_weight)
    return x.reshape(_B, _T, _D), motion_pad_mask


# speedup vs baseline: 1.1969x; 1.1969x over previous
import functools

import jax
import jax.numpy as jnp
from jax import lax
from jax.experimental import pallas as pl
from jax.experimental.pallas import tpu as pltpu
from jax.experimental.pallas import tpu_sc as plsc

_B, _T, _D = 4096, 200, 64
_LANES = 16
_ROWS_PER_CHUNK = 2
_CTOK = _ROWS_PER_CHUNK * _T
_SUBCHUNKS = ((0, 128), (128, 128), (256, 72), (328, 72))


@functools.cache
def _build_kernel():
    info = plsc.get_sparse_core_info()
    nc, ns = info.num_cores, info.num_subcores
    nw = nc * ns
    rows_per_w = _B // nw
    tok_per_w = rows_per_w * _T
    n_chunks = rows_per_w // _ROWS_PER_CHUNK
    n_super = n_chunks // 2
    mesh = plsc.VectorSubcoreMesh(core_axis_name="c", subcore_axis_name="s")

    @functools.partial(
        pl.kernel,
        mesh=mesh,
        compiler_params=pltpu.CompilerParams(use_tc_tiling_on_sc=False),
        out_type=jax.ShapeDtypeStruct((_B * _T, _D), jnp.float32),
        scratch_types=[
            pltpu.VMEM((tok_per_w,), jnp.int32),
            pltpu.VMEM((_T, _D), jnp.float32),
            pltpu.VMEM((_CTOK, _D), jnp.float32),
            pltpu.VMEM((_CTOK, _D), jnp.float32),
            pltpu.SemaphoreType.DMA,
            pltpu.SemaphoreType.DMA,
            pltpu.SemaphoreType.DMA,
            pltpu.SemaphoreType.DMA,
        ],
    )
    def k(codes_hbm, emb_hbm, pos_hbm, out_hbm, idx_v, pos_v, buf0, buf1,
          g0, g1, s0, s1):
        wid = lax.axis_index("s") * nc + lax.axis_index("c")
        tok_base = pl.multiple_of(wid * tok_per_w, 8)
        pltpu.sync_copy(codes_hbm.at[pl.ds(tok_base, tok_per_w)], idx_v)
        pltpu.sync_copy(pos_hbm, pos_v)

        def gfire(c, buf, sem):
            off = c * _CTOK
            for o, n in _SUBCHUNKS:
                pltpu.async_copy(
                    emb_hbm.at[idx_v.at[pl.ds(off + o, n)]],
                    buf.at[pl.ds(o, n)], sem)

        def gwait(c, buf, sem):
            off = c * _CTOK
            for o, n in _SUBCHUNKS:
                pltpu.make_async_copy(
                    emb_hbm.at[idx_v.at[pl.ds(off + o, n)]],
                    buf.at[pl.ds(o, n)], sem).wait()

        def sfire(c, buf, sem):
            pltpu.async_copy(
                buf, out_hbm.at[pl.ds(tok_base + c * _CTOK, _CTOK)], sem)

        def swait(c, buf, sem):
            pltpu.make_async_copy(
                buf, out_hbm.at[pl.ds(tok_base + c * _CTOK, _CTOK)],
                sem).wait()

        def add_chunk(buf):
            def body(t, carry):
                for dt in range(2):
                    tt = t * 2 + dt
                    for j in range(_D // _LANES):
                        sl = pl.ds(j * _LANES, _LANES)
                        pv = pos_v[tt, sl]
                        buf[tt, sl] = buf[tt, sl] + pv
                        buf[_T + tt, sl] = buf[_T + tt, sl] + pv
                return carry
            lax.fori_loop(0, _T // 2, body, 0)

        gfire(0, buf0, g0)

        def super_body(i, carry):
            c0 = 2 * i
            c1 = 2 * i + 1

            @pl.when(i >= 1)
            def _():
                swait(c0 - 1, buf1, s1)

            gfire(c1, buf1, g1)
            gwait(c0, buf0, g0)
            add_chunk(buf0)
            sfire(c0, buf0, s0)
            gwait(c1, buf1, g1)
            add_chunk(buf1)
            sfire(c1, buf1, s1)

            @pl.when(i < n_super - 1)
            def _():
                swait(c0, buf0, s0)
                gfire(c0 + 2, buf0, g0)

            return carry

        lax.fori_loop(0, n_super, super_body, 0)
        swait(n_chunks - 2, buf0, s0)
        swait(n_chunks - 1, buf1, s1)

    return k


def kernel(motion_codes, motion_pad_mask, emb_weight, pos_weight):
    codes = motion_codes.reshape(-1).astype(jnp.int32)
    x = _build_kernel()(codes, emb_weight, pos_weight)
    return x.reshape(_B, _T, _D), motion_pad_mask
